# baseline (speedup 1.0000x reference)
import functools

import jax
import jax.numpy as jnp
from jax.experimental import pallas as pl
from jax.experimental.pallas import tpu as pltpu

_CHUNK = 256
_LOG2E = 1.4426950408889634


def _gat_kernel(x_ref, w_ref, a_ref, adj_ref, out_ref, hid_ref, s1_ref, s2_ref):
    i = pl.program_id(0)

    @pl.when(i == 0)
    def _init():
        hid = jnp.dot(x_ref[...], w_ref[...], preferred_element_type=jnp.float32)
        hid_ref[...] = hid
        d = w_ref.shape[1]
        a1 = a_ref[:d, :]
        a2 = a_ref[d:, :]
        s1_ref[...] = jnp.dot(hid, a1, preferred_element_type=jnp.float32)
        s2_ref[...] = jax.lax.dot_general(
            a2, hid, (((0,), (1,)), ((), ())),
            preferred_element_type=jnp.float32)

    @pl.when(i > 0)
    def _chunk():
        k = i - 1
        s1_blk = s1_ref[pl.ds(k * _CHUNK, _CHUNK), :]
        logits = s1_blk + s2_ref[...]
        m = jnp.where(logits >= 0.0, -_LOG2E, -0.2 * _LOG2E)
        e = jnp.where(adj_ref[...] != 0, jnp.exp2(logits * m), 0.0)
        rowsum = jnp.sum(e, axis=1, keepdims=True)
        h = jnp.dot(e, hid_ref[...], preferred_element_type=jnp.float32)
        hp = h / (rowsum + 1e-9)
        out_ref[...] = jnp.where(
            hp > 0.0, hp, jnp.exp(jnp.minimum(hp, 0.0)) - 1.0)


@jax.jit
def kernel(x, adj, W, a):
    n, din = x.shape
    dout = W.shape[1]
    grid = n // _CHUNK + 1
    chunk_ix = lambda i: (jnp.maximum(i - 1, 0), 0)
    return pl.pallas_call(
        _gat_kernel,
        grid=(grid,),
        in_specs=[
            pl.BlockSpec((n, din), lambda i: (0, 0)),
            pl.BlockSpec((din, dout), lambda i: (0, 0)),
            pl.BlockSpec((2 * dout, 1), lambda i: (0, 0)),
            pl.BlockSpec((_CHUNK, n), chunk_ix),
        ],
        out_specs=pl.BlockSpec((_CHUNK, dout), chunk_ix),
        out_shape=jax.ShapeDtypeStruct((n, dout), jnp.float32),
        scratch_shapes=[
            pltpu.VMEM((n, dout), jnp.float32),
            pltpu.VMEM((n, 1), jnp.float32),
            pltpu.VMEM((1, n), jnp.float32),
        ],
    )(x, W, a, adj)


# speedup vs baseline: 1.2074x; 1.2074x over previous
import functools

import jax
import jax.numpy as jnp
from jax.experimental import pallas as pl
from jax.experimental.pallas import tpu as pltpu

_BLK = 512
_LOG2E = 1.4426950408889634


def _gat_kernel(x_ref, w_ref, a_ref, adj_ref, out_ref, hid_ref, s1_ref, s2_ref):
    i = pl.program_id(0)

    @pl.when(i == 0)
    def _init():
        hid = jnp.dot(x_ref[...], w_ref[...], preferred_element_type=jnp.float32)
        hid_ref[...] = hid
        d = w_ref.shape[1]
        a1 = a_ref[:d, :]
        a2 = a_ref[d:, :]
        s1_ref[...] = jnp.dot(hid, a1, preferred_element_type=jnp.float32)
        s2_ref[...] = jax.lax.dot_general(
            a2, hid, (((0,), (1,)), ((), ())),
            preferred_element_type=jnp.float32)

    s1_blk = s1_ref[pl.ds(i * _BLK, _BLK), :]
    logits = s1_blk + s2_ref[...]
    m = jnp.where(logits >= 0.0, -_LOG2E, -0.2 * _LOG2E)
    e = jnp.where(adj_ref[...] != 0, jnp.exp2(logits * m), 0.0)
    rowsum = jnp.sum(e, axis=1, keepdims=True)
    h = jnp.dot(e, hid_ref[...], preferred_element_type=jnp.float32)
    hp = h / (rowsum + 1e-9)
    out_ref[...] = jnp.where(hp > 0.0, hp, jnp.exp(jnp.minimum(hp, 0.0)) - 1.0)


@jax.jit
def kernel(x, adj, W, a):
    n, din = x.shape
    dout = W.shape[1]
    grid = n // _BLK
    return pl.pallas_call(
        _gat_kernel,
        grid=(grid,),
        in_specs=[
            pl.BlockSpec((n, din), lambda i: (0, 0)),
            pl.BlockSpec((din, dout), lambda i: (0, 0)),
            pl.BlockSpec((2 * dout, 1), lambda i: (0, 0)),
            pl.BlockSpec((_BLK, n), lambda i: (i, 0)),
        ],
        out_specs=pl.BlockSpec((_BLK, dout), lambda i: (i, 0)),
        out_shape=jax.ShapeDtypeStruct((n, dout), jnp.float32),
        scratch_shapes=[
            pltpu.VMEM((n, dout), jnp.float32),
            pltpu.VMEM((n, 1), jnp.float32),
            pltpu.VMEM((1, n), jnp.float32),
        ],
    )(x, W, a, adj)
